# baseline (speedup 1.0000x reference)
import functools

import jax
import jax.numpy as jnp
from jax import lax
from jax.experimental import pallas as pl
from jax.experimental.pallas import tpu as pltpu
from jax.experimental.pallas import tpu_sc as plsc

KNN = 16
_ROWS = 1024


def _topk_body(p_ref, q_ref, cols_ref, dist_ref, nbg_ref):
    b = pl.program_id(0)
    p = p_ref[0]
    q = q_ref[0]
    rows, n = p.shape[0], q.shape[1]
    d0 = p[:, 0:1] - q[0:1, :]
    d1 = p[:, 1:2] - q[1:2, :]
    d2 = p[:, 2:3] - q[2:3, :]
    dsq = d0 * d0 + d1 * d1 + d2 * d2
    cols = cols_ref[...]
    inf = jnp.float32(jnp.inf)
    bigj = jnp.float32(n)
    dists, nbs = [], []
    work = dsq
    for _ in range(KNN):
        m = jnp.min(work, axis=1, keepdims=True)
        j = jnp.min(jnp.where(work == m, cols, bigj), axis=1, keepdims=True)
        work = jnp.where(cols == j, inf, work)
        dists.append(m)
        nbs.append(j)
    dist_ref[0] = jnp.concatenate(dists, axis=1)
    nbg_ref[0] = jnp.concatenate(nbs, axis=1).astype(jnp.int32) + b * n


def _topk_tc(point, point_t):
    b, n, _ = point.shape
    colsf = jnp.arange(n, dtype=jnp.float32).reshape(1, n)
    return pl.pallas_call(
        _topk_body,
        grid=(b, n // _ROWS),
        in_specs=[
            pl.BlockSpec((1, _ROWS, 3), lambda i, r: (i, r, 0)),
            pl.BlockSpec((1, 3, n), lambda i, r: (i, 0, 0)),
            pl.BlockSpec((1, n), lambda i, r: (0, 0)),
        ],
        out_specs=[
            pl.BlockSpec((1, _ROWS, KNN), lambda i, r: (i, r, 0)),
            pl.BlockSpec((1, _ROWS, KNN), lambda i, r: (i, r, 0)),
        ],
        out_shape=[
            jax.ShapeDtypeStruct((b, n, KNN), jnp.float32),
            jax.ShapeDtypeStruct((b, n, KNN), jnp.int32),
        ],
    )(point, point_t, colsf)


def _sc_gather(attr_flat, nbg_flat, idxf):
    tot = nbg_flat.shape[0]
    nrow = idxf.shape[0]
    da = attr_flat.shape[1]
    info = plsc.get_sparse_core_info()
    nw = info.num_cores * info.num_subcores
    per_w = tot // nw
    q_per_w = nrow // nw
    ch = 256
    n_ch = per_w // ch
    mesh = plsc.VectorSubcoreMesh(core_axis_name="c", subcore_axis_name="s")

    @functools.partial(
        pl.kernel,
        mesh=mesh,
        compiler_params=pltpu.CompilerParams(needs_layout_passes=False),
        out_type=[
            jax.ShapeDtypeStruct((tot, da), jnp.float32),
            jax.ShapeDtypeStruct((tot,), jnp.float32),
        ],
        scratch_types=[
            pltpu.VMEM((per_w,), jnp.int32),
            pltpu.VMEM((ch, da), jnp.float32),
            pltpu.VMEM((ch, da), jnp.float32),
            pltpu.VMEM((nrow,), jnp.float32),
            pltpu.VMEM((per_w,), jnp.float32),
            pltpu.SemaphoreType.DMA,
            pltpu.SemaphoreType.DMA,
            pltpu.SemaphoreType.DMA,
            pltpu.SemaphoreType.DMA,
            pltpu.SemaphoreType.DMA,
        ],
    )
    def k(table, nbg, idx_hbm, attr_out, idd_out,
          idx_v, r0, r1, idxf_v, idd_v, g0, g1, s0, s1, isem):
        c = lax.axis_index("c")
        s = lax.axis_index("s")
        wid = s * info.num_cores + c
        base = wid * per_w
        pltpu.sync_copy(nbg.at[pl.ds(base, per_w)], idx_v)
        icopy = pltpu.async_copy(idx_hbm, idxf_v, isem)

        bufs, gs, ss = (r0, r1), (g0, g1), (s0, s1)
        gcp = [None] * n_ch
        scp = [None] * n_ch
        for i in range(n_ch):
            bi = i % 2
            if i >= 2:
                scp[i - 2].wait()
            gcp[i] = pltpu.async_copy(
                table.at[idx_v.at[pl.ds(i * ch, ch)]], bufs[bi], gs[bi])
            if i >= 1:
                gcp[i - 1].wait()
                scp[i - 1] = pltpu.async_copy(
                    bufs[1 - bi], attr_out.at[pl.ds(base + (i - 1) * ch, ch)],
                    ss[1 - bi])
        gcp[n_ch - 1].wait()
        scp[n_ch - 1] = pltpu.async_copy(
            bufs[(n_ch - 1) % 2],
            attr_out.at[pl.ds(base + (n_ch - 1) * ch, ch)],
            ss[(n_ch - 1) % 2])

        icopy.wait()
        qbase = wid * q_per_w

        def qstep(qi, carry):
            nb16 = idx_v[pl.ds(qi * KNN, KNN)]
            vals = plsc.load_gather(idxf_v, [nb16])
            csplat = jnp.full((KNN,), qbase + qi, dtype=jnp.int32)
            cvals = plsc.load_gather(idxf_v, [csplat])
            idd_v[pl.ds(qi * KNN, KNN)] = jnp.abs(vals - cvals)
            return carry

        lax.fori_loop(0, q_per_w, qstep, 0)
        pltpu.sync_copy(idd_v, idd_out.at[pl.ds(base, per_w)])
        scp[n_ch - 2].wait()
        scp[n_ch - 1].wait()

    return k(attr_flat, nbg_flat, idxf)


def kernel(point, index, attr):
    b, n, _ = point.shape
    da = attr.shape[-1]
    point_t = jnp.transpose(point, (0, 2, 1))
    dist, nbg = _topk_tc(point, point_t)
    attr_flat = attr.reshape(b * n, da)
    nbg_flat = nbg.reshape(b * n * KNN)
    idxf = index.reshape(b * n).astype(jnp.float32)
    attr_rows, idd = _sc_gather(attr_flat, nbg_flat, idxf)
    return (
        dist.reshape(b, n, KNN, 1),
        idd.reshape(b, n, KNN, 1),
        attr_rows.reshape(b, n, KNN, da),
    )


# speedup vs baseline: 1.0015x; 1.0015x over previous
import functools

import jax
import jax.numpy as jnp
from jax import lax
from jax.experimental import pallas as pl
from jax.experimental.pallas import tpu as pltpu
from jax.experimental.pallas import tpu_sc as plsc

KNN = 16
_ROWS = 512


def _topk_body(p_ref, q_ref, cols_ref, dist_ref, nbg_ref):
    b = pl.program_id(0)
    p = p_ref[0]
    q = q_ref[0]
    rows, n = p.shape[0], q.shape[1]
    d0 = p[:, 0:1] - q[0:1, :]
    d1 = p[:, 1:2] - q[1:2, :]
    d2 = p[:, 2:3] - q[2:3, :]
    dsq = d0 * d0 + d1 * d1 + d2 * d2
    cols = cols_ref[...]
    inf = jnp.float32(jnp.inf)
    bigj = jnp.float32(n)
    dists, nbs = [], []
    work = dsq
    for _ in range(KNN):
        m = jnp.min(work, axis=1, keepdims=True)
        j = jnp.min(jnp.where(work == m, cols, bigj), axis=1, keepdims=True)
        work = jnp.where(cols == j, inf, work)
        dists.append(m)
        nbs.append(j)
    dist_ref[0] = jnp.concatenate(dists, axis=1)
    nbg_ref[0] = jnp.concatenate(nbs, axis=1).astype(jnp.int32) + b * n


def _topk_tc(point, point_t):
    b, n, _ = point.shape
    colsf = jnp.arange(n, dtype=jnp.float32).reshape(1, n)
    return pl.pallas_call(
        _topk_body,
        grid=(b, n // _ROWS),
        in_specs=[
            pl.BlockSpec((1, _ROWS, 3), lambda i, r: (i, r, 0)),
            pl.BlockSpec((1, 3, n), lambda i, r: (i, 0, 0)),
            pl.BlockSpec((1, n), lambda i, r: (0, 0)),
        ],
        out_specs=[
            pl.BlockSpec((1, _ROWS, KNN), lambda i, r: (i, r, 0)),
            pl.BlockSpec((1, _ROWS, KNN), lambda i, r: (i, r, 0)),
        ],
        out_shape=[
            jax.ShapeDtypeStruct((b, n, KNN), jnp.float32),
            jax.ShapeDtypeStruct((b, n, KNN), jnp.int32),
        ],
    )(point, point_t, colsf)


def _sc_gather(attr_flat, nbg_flat, idxf):
    tot = nbg_flat.shape[0]
    nrow = idxf.shape[0]
    da = attr_flat.shape[1]
    info = plsc.get_sparse_core_info()
    nw = info.num_cores * info.num_subcores
    per_w = tot // nw
    q_per_w = nrow // nw
    ch = 256
    n_ch = per_w // ch
    mesh = plsc.VectorSubcoreMesh(core_axis_name="c", subcore_axis_name="s")

    @functools.partial(
        pl.kernel,
        mesh=mesh,
        compiler_params=pltpu.CompilerParams(needs_layout_passes=False),
        out_type=[
            jax.ShapeDtypeStruct((tot, da), jnp.float32),
            jax.ShapeDtypeStruct((tot,), jnp.float32),
        ],
        scratch_types=[
            pltpu.VMEM((per_w,), jnp.int32),
            pltpu.VMEM((ch, da), jnp.float32),
            pltpu.VMEM((ch, da), jnp.float32),
            pltpu.VMEM((nrow,), jnp.float32),
            pltpu.VMEM((per_w,), jnp.float32),
            pltpu.SemaphoreType.DMA,
            pltpu.SemaphoreType.DMA,
            pltpu.SemaphoreType.DMA,
            pltpu.SemaphoreType.DMA,
            pltpu.SemaphoreType.DMA,
        ],
    )
    def k(table, nbg, idx_hbm, attr_out, idd_out,
          idx_v, r0, r1, idxf_v, idd_v, g0, g1, s0, s1, isem):
        c = lax.axis_index("c")
        s = lax.axis_index("s")
        wid = s * info.num_cores + c
        base = wid * per_w
        pltpu.sync_copy(nbg.at[pl.ds(base, per_w)], idx_v)
        icopy = pltpu.async_copy(idx_hbm, idxf_v, isem)

        bufs, gs, ss = (r0, r1), (g0, g1), (s0, s1)
        gcp = [None] * n_ch
        scp = [None] * n_ch
        for i in range(n_ch):
            bi = i % 2
            if i >= 2:
                scp[i - 2].wait()
            gcp[i] = pltpu.async_copy(
                table.at[idx_v.at[pl.ds(i * ch, ch)]], bufs[bi], gs[bi])
            if i >= 1:
                gcp[i - 1].wait()
                scp[i - 1] = pltpu.async_copy(
                    bufs[1 - bi], attr_out.at[pl.ds(base + (i - 1) * ch, ch)],
                    ss[1 - bi])
        gcp[n_ch - 1].wait()
        scp[n_ch - 1] = pltpu.async_copy(
            bufs[(n_ch - 1) % 2],
            attr_out.at[pl.ds(base + (n_ch - 1) * ch, ch)],
            ss[(n_ch - 1) % 2])

        icopy.wait()
        qbase = wid * q_per_w

        def qstep(qi, carry):
            nb16 = idx_v[pl.ds(qi * KNN, KNN)]
            vals = plsc.load_gather(idxf_v, [nb16])
            csplat = jnp.full((KNN,), qbase + qi, dtype=jnp.int32)
            cvals = plsc.load_gather(idxf_v, [csplat])
            idd_v[pl.ds(qi * KNN, KNN)] = jnp.abs(vals - cvals)
            return carry

        lax.fori_loop(0, q_per_w, qstep, 0)
        pltpu.sync_copy(idd_v, idd_out.at[pl.ds(base, per_w)])
        scp[n_ch - 2].wait()
        scp[n_ch - 1].wait()

    return k(attr_flat, nbg_flat, idxf)


def kernel(point, index, attr):
    b, n, _ = point.shape
    da = attr.shape[-1]
    point_t = jnp.transpose(point, (0, 2, 1))
    dist, nbg = _topk_tc(point, point_t)
    attr_flat = attr.reshape(b * n, da)
    nbg_flat = nbg.reshape(b * n * KNN)
    idxf = index.reshape(b * n).astype(jnp.float32)
    attr_rows, idd = _sc_gather(attr_flat, nbg_flat, idxf)
    return (
        dist.reshape(b, n, KNN, 1),
        idd.reshape(b, n, KNN, 1),
        attr_rows.reshape(b, n, KNN, da),
    )


# speedup vs baseline: 1.0351x; 1.0336x over previous
import functools

import jax
import jax.numpy as jnp
from jax import lax
from jax.experimental import pallas as pl
from jax.experimental.pallas import tpu as pltpu
from jax.experimental.pallas import tpu_sc as plsc

KNN = 16
_ROWS = 512


def _topk_body(p_ref, q_ref, cols_ref, dist_ref, nbg_ref):
    b = pl.program_id(0)
    p = p_ref[0]
    q = q_ref[0]
    rows, n = p.shape[0], q.shape[1]
    p2 = p[:, 0:1] * p[:, 0:1] + p[:, 1:2] * p[:, 1:2] + p[:, 2:3] * p[:, 2:3]
    q2 = q[0:1, :] * q[0:1, :] + q[1:2, :] * q[1:2, :] + q[2:3, :] * q[2:3, :]
    mm = jax.lax.dot_general((-2.0 * p), q, (((1,), (0,)), ((), ())),
                             preferred_element_type=jnp.float32)
    dsq = mm + p2 + q2
    cols = cols_ref[...]
    inf = jnp.float32(jnp.inf)
    bigj = jnp.float32(n)
    dists, nbs = [], []
    work = dsq
    for _ in range(KNN):
        m = jnp.min(work, axis=1, keepdims=True)
        j = jnp.min(jnp.where(work == m, cols, bigj), axis=1, keepdims=True)
        work = jnp.where(cols == j, inf, work)
        dists.append(m)
        nbs.append(j)
    dist_ref[0] = jnp.concatenate(dists, axis=1)
    nbg_ref[0] = jnp.concatenate(nbs, axis=1).astype(jnp.int32) + b * n


def _topk_tc(point, point_t):
    b, n, _ = point.shape
    colsf = jnp.arange(n, dtype=jnp.float32).reshape(1, n)
    return pl.pallas_call(
        _topk_body,
        grid=(b, n // _ROWS),
        in_specs=[
            pl.BlockSpec((1, _ROWS, 3), lambda i, r: (i, r, 0)),
            pl.BlockSpec((1, 3, n), lambda i, r: (i, 0, 0)),
            pl.BlockSpec((1, n), lambda i, r: (0, 0)),
        ],
        out_specs=[
            pl.BlockSpec((1, _ROWS, KNN), lambda i, r: (i, r, 0)),
            pl.BlockSpec((1, _ROWS, KNN), lambda i, r: (i, r, 0)),
        ],
        out_shape=[
            jax.ShapeDtypeStruct((b, n, KNN), jnp.float32),
            jax.ShapeDtypeStruct((b, n, KNN), jnp.int32),
        ],
    )(point, point_t, colsf)


def _sc_gather(attr_flat, nbg_flat, idxf):
    tot = nbg_flat.shape[0]
    nrow = idxf.shape[0]
    da = attr_flat.shape[1]
    info = plsc.get_sparse_core_info()
    nw = info.num_cores * info.num_subcores
    per_w = tot // nw
    q_per_w = nrow // nw
    ch = 256
    n_ch = per_w // ch
    mesh = plsc.VectorSubcoreMesh(core_axis_name="c", subcore_axis_name="s")

    @functools.partial(
        pl.kernel,
        mesh=mesh,
        compiler_params=pltpu.CompilerParams(needs_layout_passes=False),
        out_type=[
            jax.ShapeDtypeStruct((tot, da), jnp.float32),
            jax.ShapeDtypeStruct((tot,), jnp.float32),
        ],
        scratch_types=[
            pltpu.VMEM((per_w,), jnp.int32),
            pltpu.VMEM((ch, da), jnp.float32),
            pltpu.VMEM((ch, da), jnp.float32),
            pltpu.VMEM((nrow,), jnp.float32),
            pltpu.VMEM((per_w,), jnp.float32),
            pltpu.SemaphoreType.DMA,
            pltpu.SemaphoreType.DMA,
            pltpu.SemaphoreType.DMA,
            pltpu.SemaphoreType.DMA,
            pltpu.SemaphoreType.DMA,
        ],
    )
    def k(table, nbg, idx_hbm, attr_out, idd_out,
          idx_v, r0, r1, idxf_v, idd_v, g0, g1, s0, s1, isem):
        c = lax.axis_index("c")
        s = lax.axis_index("s")
        wid = s * info.num_cores + c
        base = wid * per_w
        pltpu.sync_copy(nbg.at[pl.ds(base, per_w)], idx_v)
        icopy = pltpu.async_copy(idx_hbm, idxf_v, isem)

        bufs, gs, ss = (r0, r1), (g0, g1), (s0, s1)
        gcp = [None] * n_ch
        scp = [None] * n_ch
        for i in range(n_ch):
            bi = i % 2
            if i >= 2:
                scp[i - 2].wait()
            gcp[i] = pltpu.async_copy(
                table.at[idx_v.at[pl.ds(i * ch, ch)]], bufs[bi], gs[bi])
            if i >= 1:
                gcp[i - 1].wait()
                scp[i - 1] = pltpu.async_copy(
                    bufs[1 - bi], attr_out.at[pl.ds(base + (i - 1) * ch, ch)],
                    ss[1 - bi])
        gcp[n_ch - 1].wait()
        scp[n_ch - 1] = pltpu.async_copy(
            bufs[(n_ch - 1) % 2],
            attr_out.at[pl.ds(base + (n_ch - 1) * ch, ch)],
            ss[(n_ch - 1) % 2])

        icopy.wait()
        qbase = wid * q_per_w

        def qstep(qi, carry):
            nb16 = idx_v[pl.ds(qi * KNN, KNN)]
            vals = plsc.load_gather(idxf_v, [nb16])
            csplat = jnp.full((KNN,), qbase + qi, dtype=jnp.int32)
            cvals = plsc.load_gather(idxf_v, [csplat])
            idd_v[pl.ds(qi * KNN, KNN)] = jnp.abs(vals - cvals)
            return carry

        lax.fori_loop(0, q_per_w, qstep, 0)
        pltpu.sync_copy(idd_v, idd_out.at[pl.ds(base, per_w)])
        scp[n_ch - 2].wait()
        scp[n_ch - 1].wait()

    return k(attr_flat, nbg_flat, idxf)


def kernel(point, index, attr):
    b, n, _ = point.shape
    da = attr.shape[-1]
    point_t = jnp.transpose(point, (0, 2, 1))
    dist, nbg = _topk_tc(point, point_t)
    attr_flat = attr.reshape(b * n, da)
    nbg_flat = nbg.reshape(b * n * KNN)
    idxf = index.reshape(b * n).astype(jnp.float32)
    attr_rows, idd = _sc_gather(attr_flat, nbg_flat, idxf)
    return (
        dist.reshape(b, n, KNN, 1),
        idd.reshape(b, n, KNN, 1),
        attr_rows.reshape(b, n, KNN, da),
    )
